# baseline (speedup 1.0000x reference)
import functools

import jax
import jax.numpy as jnp
from jax import lax
from jax.experimental import pallas as pl
from jax.experimental.pallas import tpu as pltpu
from jax.experimental.pallas import tpu_sc as plsc

BATCH = 16384
DIM = 64
MARGIN = 1.0

NW = 32
ROWS_PER_DMA = 128

ENT_LOOKUPS = 4 * BATCH
REL_LOOKUPS = 2 * BATCH
ENT_PER_W = ENT_LOOKUPS // NW
REL_PER_W = REL_LOOKUPS // NW
ENT_DMAS = ENT_PER_W // ROWS_PER_DMA
REL_DMAS = REL_PER_W // ROWS_PER_DMA


def _sc_gather_body(ent_t, rel_t, eidx, ridx, ent_out, rel_out,
                    eidx_v, ridx_v, rows_v, sem_a, sem_b):
    wid = lax.axis_index("s") * 2 + lax.axis_index("c")
    pltpu.sync_copy(eidx.at[pl.ds(wid * ENT_DMAS, ENT_DMAS)], eidx_v)
    pltpu.sync_copy(ridx.at[pl.ds(wid * REL_DMAS, REL_DMAS)], ridx_v)

    sems = (sem_a, sem_b)

    def run(table, idx_v, out, n_dmas, out_base):
        pend = [None, None]
        pend[0] = pltpu.async_copy(table.at[idx_v.at[0]], rows_v.at[0], sems[0])
        for j in range(n_dmas):
            if j + 1 < n_dmas:
                b = (j + 1) % 2
                pend[b] = pltpu.async_copy(
                    table.at[idx_v.at[j + 1]], rows_v.at[b], sems[b])
            pend[j % 2].wait()
            pltpu.sync_copy(
                rows_v.at[j % 2],
                out.at[pl.ds(out_base + j * ROWS_PER_DMA, ROWS_PER_DMA)])

    run(ent_t, eidx_v, ent_out, ENT_DMAS, wid * ENT_PER_W)
    run(rel_t, ridx_v, rel_out, REL_DMAS, wid * REL_PER_W)


def _make_sc_gather():
    mesh = plsc.VectorSubcoreMesh(core_axis_name="c", subcore_axis_name="s")
    return functools.partial(
        pl.kernel, mesh=mesh,
        compiler_params=pltpu.CompilerParams(use_tc_tiling_on_sc=False),
        out_type=[
            jax.ShapeDtypeStruct((ENT_LOOKUPS, DIM), jnp.float32),
            jax.ShapeDtypeStruct((REL_LOOKUPS, DIM), jnp.float32),
        ],
        scratch_types=[
            pltpu.VMEM((ENT_DMAS, ROWS_PER_DMA), jnp.int32),
            pltpu.VMEM((REL_DMAS, ROWS_PER_DMA), jnp.int32),
            pltpu.VMEM((2, ROWS_PER_DMA, DIM), jnp.float32),
            pltpu.SemaphoreType.DMA,
            pltpu.SemaphoreType.DMA,
        ],
    )(_sc_gather_body)


_sc_gather = _make_sc_gather()

CB = 2048
NBLK = BATCH // CB


def _score_body(ph, pt, pr, nh, nt, nr, out):
    k = pl.program_id(0)

    def score(h_ref, t_ref, r_ref):
        h = h_ref[...]
        t = t_ref[...]
        r = r_ref[...]
        hn = h / jnp.sqrt(jnp.sum(h * h, axis=1, keepdims=True))
        tn = t / jnp.sqrt(jnp.sum(t * t, axis=1, keepdims=True))
        diff = hn + r - tn
        return jnp.sqrt(jnp.sum(diff * diff, axis=1))

    ps = score(ph, pt, pr)
    ns = score(nh, nt, nr)
    part = jnp.sum(jnp.maximum(MARGIN + ps - ns, 0.0)).reshape(1, 1)
    prev = jnp.where(k == 0, jnp.zeros((1, 1), jnp.float32), out[...])
    total = prev + part
    out[...] = jnp.where(k == NBLK - 1, total / BATCH, total)


def _tc_score(ent_rows, rel_rows):
    def blk(off):
        return pl.BlockSpec((CB, DIM), lambda k, o=off: (k + o, 0))

    out = pl.pallas_call(
        _score_body,
        grid=(NBLK,),
        in_specs=[blk(0), blk(NBLK), blk(0), blk(2 * NBLK), blk(3 * NBLK),
                  blk(NBLK)],
        out_specs=pl.BlockSpec((1, 1), lambda k: (0, 0)),
        out_shape=jax.ShapeDtypeStruct((1, 1), jnp.float32),
    )(ent_rows, ent_rows, rel_rows, ent_rows, ent_rows, rel_rows)
    return out.reshape(())


def kernel(pos_x, neg_x, ent_table, rel_table):
    ent_idx = jnp.concatenate(
        [pos_x[:, 0], pos_x[:, 1], neg_x[:, 0], neg_x[:, 1]]
    ).reshape(ENT_LOOKUPS // ROWS_PER_DMA, ROWS_PER_DMA)
    rel_idx = jnp.concatenate(
        [pos_x[:, 2], neg_x[:, 2]]
    ).reshape(REL_LOOKUPS // ROWS_PER_DMA, ROWS_PER_DMA)
    ent_rows, rel_rows = _sc_gather(ent_table, rel_table, ent_idx, rel_idx)
    return _tc_score(ent_rows, rel_rows)
